# baseline (speedup 1.0000x reference)
import functools

import jax
import jax.numpy as jnp
from jax import lax
from jax.experimental import pallas as pl
from jax.experimental.pallas import tpu as pltpu
from jax.experimental.pallas import tpu_sc as plsc

_M = 1024
_D = 256
_BLK = 2304
_K = 2


def _vq_kernel(nblocks, n_rows, last, x_ref, embt_ref, embt_bf_ref,
               psum_in_ref, hist_in_ref,
               inds_ref, psum_out_ref, hist_out_ref, cp_ref, pp_ref,
               e2_ref, psum_ref, hist_ref, riota_ref):
    i = pl.program_id(0)

    @pl.when(i == 0)
    def _init():
        embt = embt_ref[...]
        e2_ref[...] = jnp.sum(embt * embt, axis=0, keepdims=True)
        psum_ref[...] = psum_in_ref[...]
        hist_ref[...] = hist_in_ref[...]
        iota_i = jax.lax.broadcasted_iota(jnp.int32, riota_ref.shape, 1)
        riota_ref[...] = (_M - iota_i).astype(jnp.float32)

    x = x_ref[...]
    x2 = jnp.sum(x * x, axis=1, keepdims=True)
    s = jnp.dot(x.astype(jnp.bfloat16), embt_bf_ref[...],
                preferred_element_type=jnp.float32)
    dmap = 2.0 * s - (e2_ref[...] + x2)

    m = jnp.max(dmap, axis=1, keepdims=True)
    masked = jnp.where(dmap == m, riota_ref[...], 0.0)
    r = jnp.max(masked, axis=1, keepdims=True)
    k = (float(_M) - r).astype(jnp.int32)
    inds_ref[...] = k

    p = jnp.exp(dmap - m)
    probs = p / jnp.sum(p, axis=1, keepdims=True)
    psum_ref[...] += jnp.sum(probs, axis=0, keepdims=True)

    ohf = jnp.where(masked == r, 1.0, 0.0)
    hist_ref[...] += jnp.sum(ohf, axis=0, keepdims=True)

    @pl.when(i == nblocks - 1)
    def _finish():
        psum_out_ref[...] = psum_ref[...]
        hist_out_ref[...] = hist_ref[...]
        if last:
            inv_n = 1.0 / n_rows
            hp = hist_ref[...] * inv_n
            cp_ref[...] = -jnp.sum(hp * (jnp.log2(hp + 1e-10)), axis=1,
                                   keepdims=True)
            ap = psum_ref[...] * inv_n
            pp_ref[...] = -jnp.sum(ap * (jnp.log2(ap + 1e-10)), axis=1,
                                   keepdims=True)
        else:
            cp_ref[...] = jnp.zeros_like(cp_ref)
            pp_ref[...] = jnp.zeros_like(pp_ref)


def _tc_part(x_part, embt, embt_bf, psum_in, hist_in, n_total, last):
    np_rows = x_part.shape[0]
    nblocks = np_rows // _BLK
    return pl.pallas_call(
        functools.partial(_vq_kernel, nblocks, float(n_total), last),
        grid=(nblocks,),
        in_specs=[
            pl.BlockSpec((_BLK, _D), lambda i: (i, 0)),
            pl.BlockSpec((_D, _M), lambda i: (0, 0)),
            pl.BlockSpec((_D, _M), lambda i: (0, 0)),
            pl.BlockSpec((1, _M), lambda i: (0, 0)),
            pl.BlockSpec((1, _M), lambda i: (0, 0)),
        ],
        out_specs=[
            pl.BlockSpec((_BLK, 1), lambda i: (i, 0)),
            pl.BlockSpec((1, _M), lambda i: (0, 0)),
            pl.BlockSpec((1, _M), lambda i: (0, 0)),
            pl.BlockSpec((1, 1), lambda i: (0, 0)),
            pl.BlockSpec((1, 1), lambda i: (0, 0)),
        ],
        out_shape=[
            jax.ShapeDtypeStruct((np_rows, 1), jnp.int32),
            jax.ShapeDtypeStruct((1, _M), jnp.float32),
            jax.ShapeDtypeStruct((1, _M), jnp.float32),
            jax.ShapeDtypeStruct((1, 1), jnp.float32),
            jax.ShapeDtypeStruct((1, 1), jnp.float32),
        ],
        scratch_shapes=[
            pltpu.VMEM((1, _M), jnp.float32),
            pltpu.VMEM((1, _M), jnp.float32),
            pltpu.VMEM((1, _M), jnp.float32),
            pltpu.VMEM((1, _M), jnp.float32),
        ],
    )(x_part, embt, embt_bf, psum_in, hist_in)


def _sc_gather(n, chunk, table_hbm, idx_hbm, out_hbm, idx_v, rows_v, sem):
    info = plsc.get_sparse_core_info()
    nw = info.num_cores * info.num_subcores
    b_per_w = n // nw
    wid = lax.axis_index("s") * info.num_cores + lax.axis_index("c")
    base = wid * b_per_w
    for c in range(b_per_w // chunk):
        pltpu.sync_copy(idx_hbm.at[pl.ds(base + c * chunk, chunk)], idx_v)
        pltpu.async_copy(table_hbm.at[idx_v], rows_v, sem).wait()
        pltpu.sync_copy(rows_v, out_hbm.at[pl.ds(base + c * chunk, chunk)])


def _sc_part(table, inds_part, chunk):
    np_rows = inds_part.shape[0]
    mesh = plsc.VectorSubcoreMesh(core_axis_name="c", subcore_axis_name="s")
    return pl.kernel(
        functools.partial(_sc_gather, np_rows, chunk),
        mesh=mesh,
        out_type=jax.ShapeDtypeStruct((np_rows, _D), jnp.float32),
        scratch_types=[
            pltpu.VMEM((chunk,), jnp.int32),
            pltpu.VMEM((chunk, _D), jnp.float32),
            pltpu.SemaphoreType.DMA,
        ],
    )(table, inds_part)


def kernel(x, embedding):
    bsz, tsz, csz = x.shape
    n = bsz * tsz
    x_flat = x.reshape(n, csz)
    emb = embedding[0]
    embt = emb.T
    embt_bf = embt.astype(jnp.bfloat16)
    table = emb.astype(jnp.bfloat16).astype(jnp.float32)

    np_rows = n // _K
    chunk = 96
    psum = jnp.zeros((1, _M), jnp.float32)
    hist = jnp.zeros((1, _M), jnp.float32)
    inds_parts = []
    q_parts = []
    cp = pp = None
    for kpart in range(_K):
        x_part = lax.slice(x_flat, (kpart * np_rows, 0),
                           ((kpart + 1) * np_rows, _D))
        inds_p, psum, hist, cp, pp = _tc_part(
            x_part, embt, embt_bf, psum, hist, n, kpart == _K - 1)
        inds_parts.append(inds_p)
        q_parts.append(_sc_part(table, inds_p.reshape(np_rows), chunk))

    quantized = jnp.concatenate(q_parts, axis=0).reshape(bsz, tsz, csz)
    quantization_inds = jnp.concatenate(inds_parts, axis=0).reshape(bsz, tsz, 1)
    return (quantized, cp[0, 0], pp[0, 0], quantization_inds)


# speedup vs baseline: 1.2433x; 1.2433x over previous
import functools

import jax
import jax.numpy as jnp
from jax import lax
from jax.experimental import pallas as pl
from jax.experimental.pallas import tpu as pltpu
from jax.experimental.pallas import tpu_sc as plsc

_M = 1024
_D = 256
_BLK = 2304


def _vq_kernel(nblocks, n_rows, x_ref, embt_ref, embt2_bf_ref,
               inds_ref, cp_ref, pp_ref,
               e2_ref, psum_ref, hist_ref, riota_ref):
    i = pl.program_id(0)

    @pl.when(i == 0)
    def _init():
        embt = embt_ref[...]
        e2_ref[...] = jnp.sum(embt * embt, axis=0, keepdims=True)
        psum_ref[...] = jnp.zeros_like(psum_ref)
        hist_ref[...] = jnp.zeros_like(hist_ref)
        iota_i = jax.lax.broadcasted_iota(jnp.int32, riota_ref.shape, 1)
        riota_ref[...] = (_M - iota_i).astype(jnp.float32)

    x = x_ref[...]
    x2 = jnp.sum(x * x, axis=1, keepdims=True)
    s2 = jnp.dot(x.astype(jnp.bfloat16), embt2_bf_ref[...],
                 preferred_element_type=jnp.float32)
    dmap = s2 - (e2_ref[...] + x2)

    m = jnp.max(dmap, axis=1, keepdims=True)
    masked = jnp.where(dmap == m, riota_ref[...], 0.0)
    r = jnp.max(masked, axis=1, keepdims=True)
    k = (float(_M) - r).astype(jnp.int32)
    inds_ref[...] = k

    p = jnp.exp(dmap - m)
    inv = 1.0 / jnp.sum(p, axis=1, keepdims=True)
    psum_ref[...] += jnp.sum(p * inv, axis=0, keepdims=True)

    ohf = jnp.where(masked == r, 1.0, 0.0)
    hist_ref[...] += jnp.sum(ohf, axis=0, keepdims=True)

    @pl.when(i == nblocks - 1)
    def _finish():
        inv_n = 1.0 / n_rows
        hp = hist_ref[...] * inv_n
        cp_ref[...] = -jnp.sum(hp * (jnp.log2(hp + 1e-10)), axis=1, keepdims=True)
        ap = psum_ref[...] * inv_n
        pp_ref[...] = -jnp.sum(ap * (jnp.log2(ap + 1e-10)), axis=1, keepdims=True)


def _sc_gather(n, chunk, table_hbm, idx_hbm, out_hbm, idx_v, rows_v, sem):
    info = plsc.get_sparse_core_info()
    nw = info.num_cores * info.num_subcores
    b_per_w = n // nw
    wid = lax.axis_index("s") * info.num_cores + lax.axis_index("c")
    base = wid * b_per_w
    for c in range(b_per_w // chunk):
        pltpu.sync_copy(idx_hbm.at[pl.ds(base + c * chunk, chunk)], idx_v)
        pltpu.async_copy(table_hbm.at[idx_v], rows_v, sem).wait()
        pltpu.sync_copy(rows_v, out_hbm.at[pl.ds(base + c * chunk, chunk)])


def kernel(x, embedding):
    bsz, tsz, csz = x.shape
    n = bsz * tsz
    x_flat = x.reshape(n, csz)
    emb = embedding[0]
    embt = emb.T
    nblocks = n // _BLK

    inds, cp, pp = pl.pallas_call(
        functools.partial(_vq_kernel, nblocks, float(n)),
        grid=(nblocks,),
        in_specs=[
            pl.BlockSpec((_BLK, _D), lambda i: (i, 0)),
            pl.BlockSpec((_D, _M), lambda i: (0, 0)),
            pl.BlockSpec((_D, _M), lambda i: (0, 0)),
        ],
        out_specs=[
            pl.BlockSpec((_BLK, 1), lambda i: (i, 0)),
            pl.BlockSpec((1, 1), lambda i: (0, 0)),
            pl.BlockSpec((1, 1), lambda i: (0, 0)),
        ],
        out_shape=[
            jax.ShapeDtypeStruct((n, 1), jnp.int32),
            jax.ShapeDtypeStruct((1, 1), jnp.float32),
            jax.ShapeDtypeStruct((1, 1), jnp.float32),
        ],
        scratch_shapes=[
            pltpu.VMEM((1, _M), jnp.float32),
            pltpu.VMEM((1, _M), jnp.float32),
            pltpu.VMEM((1, _M), jnp.float32),
            pltpu.VMEM((1, _M), jnp.float32),
        ],
    )(x_flat, embt, (2.0 * embt).astype(jnp.bfloat16))

    table = emb.astype(jnp.bfloat16).astype(jnp.float32)
    mesh = plsc.VectorSubcoreMesh(core_axis_name="c", subcore_axis_name="s")
    chunk = 96
    q = pl.kernel(
        functools.partial(_sc_gather, n, chunk),
        mesh=mesh,
        out_type=jax.ShapeDtypeStruct((n, _D), jnp.float32),
        scratch_types=[
            pltpu.VMEM((chunk,), jnp.int32),
            pltpu.VMEM((chunk, _D), jnp.float32),
            pltpu.SemaphoreType.DMA,
        ],
    )(table, inds.reshape(n))

    quantized = q.reshape(bsz, tsz, csz)
    quantization_inds = inds.reshape(bsz, tsz, 1)
    return (quantized, cp[0, 0], pp[0, 0], quantization_inds)


# speedup vs baseline: 1.2705x; 1.0219x over previous
import functools

import jax
import jax.numpy as jnp
from jax import lax
from jax.experimental import pallas as pl
from jax.experimental.pallas import tpu as pltpu
from jax.experimental.pallas import tpu_sc as plsc

_M = 1024
_D = 256
_BLK = 2304


def _vq_kernel(nblocks, n_rows, x_ref, embt_ref, embt2_bf_ref,
               inds_ref, cp_ref, pp_ref,
               e2_ref, psum_ref, hist_ref, riota_ref):
    i = pl.program_id(0)

    @pl.when(i == 0)
    def _init():
        embt = embt_ref[...]
        e2_ref[...] = jnp.sum(embt * embt, axis=0, keepdims=True)
        psum_ref[...] = jnp.zeros_like(psum_ref)
        hist_ref[...] = jnp.zeros_like(hist_ref)
        iota_i = jax.lax.broadcasted_iota(jnp.int32, riota_ref.shape, 1)
        riota_ref[...] = (_M - iota_i).astype(jnp.float32)

    x = x_ref[...]
    x2 = jnp.sum(x * x, axis=1, keepdims=True)
    s2 = jnp.dot(x.astype(jnp.bfloat16), embt2_bf_ref[...],
                 preferred_element_type=jnp.float32)
    dmap = s2 - (e2_ref[...] + x2)

    m = jnp.max(dmap, axis=1, keepdims=True)
    masked = jnp.where(dmap == m, riota_ref[...], 0.0)
    r = jnp.max(masked, axis=1, keepdims=True)
    k = (float(_M) - r).astype(jnp.int32)
    inds_ref[...] = k

    p = jnp.exp(dmap - m)
    inv = 1.0 / jnp.sum(p, axis=1, keepdims=True)
    ones_row = jnp.ones((1, x.shape[0]), jnp.bfloat16)
    pb = (p * inv).astype(jnp.bfloat16)
    psum_ref[...] += jnp.dot(ones_row, pb,
                             preferred_element_type=jnp.float32)

    ohb = jnp.where(masked == r, 1.0, 0.0).astype(jnp.bfloat16)
    hist_ref[...] += jnp.dot(ones_row, ohb,
                             preferred_element_type=jnp.float32)

    @pl.when(i == nblocks - 1)
    def _finish():
        inv_n = 1.0 / n_rows
        hp = hist_ref[...] * inv_n
        cp_ref[...] = -jnp.sum(hp * (jnp.log2(hp + 1e-10)), axis=1, keepdims=True)
        ap = psum_ref[...] * inv_n
        pp_ref[...] = -jnp.sum(ap * (jnp.log2(ap + 1e-10)), axis=1, keepdims=True)


def _sc_gather(n, chunk, table_hbm, idx_hbm, out_hbm, idx_v, rows_v, sem):
    info = plsc.get_sparse_core_info()
    nw = info.num_cores * info.num_subcores
    b_per_w = n // nw
    wid = lax.axis_index("s") * info.num_cores + lax.axis_index("c")
    base = wid * b_per_w
    for c in range(b_per_w // chunk):
        pltpu.sync_copy(idx_hbm.at[pl.ds(base + c * chunk, chunk)], idx_v)
        pltpu.async_copy(table_hbm.at[idx_v], rows_v, sem).wait()
        pltpu.sync_copy(rows_v, out_hbm.at[pl.ds(base + c * chunk, chunk)])


def kernel(x, embedding):
    bsz, tsz, csz = x.shape
    n = bsz * tsz
    x_flat = x.reshape(n, csz)
    emb = embedding[0]
    embt = emb.T
    nblocks = n // _BLK

    inds, cp, pp = pl.pallas_call(
        functools.partial(_vq_kernel, nblocks, float(n)),
        grid=(nblocks,),
        in_specs=[
            pl.BlockSpec((_BLK, _D), lambda i: (i, 0)),
            pl.BlockSpec((_D, _M), lambda i: (0, 0)),
            pl.BlockSpec((_D, _M), lambda i: (0, 0)),
        ],
        out_specs=[
            pl.BlockSpec((_BLK, 1), lambda i: (i, 0)),
            pl.BlockSpec((1, 1), lambda i: (0, 0)),
            pl.BlockSpec((1, 1), lambda i: (0, 0)),
        ],
        out_shape=[
            jax.ShapeDtypeStruct((n, 1), jnp.int32),
            jax.ShapeDtypeStruct((1, 1), jnp.float32),
            jax.ShapeDtypeStruct((1, 1), jnp.float32),
        ],
        scratch_shapes=[
            pltpu.VMEM((1, _M), jnp.float32),
            pltpu.VMEM((1, _M), jnp.float32),
            pltpu.VMEM((1, _M), jnp.float32),
            pltpu.VMEM((1, _M), jnp.float32),
        ],
    )(x_flat, embt, (2.0 * embt).astype(jnp.bfloat16))

    table = emb.astype(jnp.bfloat16).astype(jnp.float32)
    mesh = plsc.VectorSubcoreMesh(core_axis_name="c", subcore_axis_name="s")
    chunk = 96
    q = pl.kernel(
        functools.partial(_sc_gather, n, chunk),
        mesh=mesh,
        out_type=jax.ShapeDtypeStruct((n, _D), jnp.float32),
        scratch_types=[
            pltpu.VMEM((chunk,), jnp.int32),
            pltpu.VMEM((chunk, _D), jnp.float32),
            pltpu.SemaphoreType.DMA,
        ],
    )(table, inds.reshape(n))

    quantized = q.reshape(bsz, tsz, csz)
    quantization_inds = inds.reshape(bsz, tsz, 1)
    return (quantized, cp[0, 0], pp[0, 0], quantization_inds)


# speedup vs baseline: 1.2958x; 1.0199x over previous
import functools

import jax
import jax.numpy as jnp
from jax import lax
from jax.experimental import pallas as pl
from jax.experimental.pallas import tpu as pltpu
from jax.experimental.pallas import tpu_sc as plsc

_M = 1024
_D = 256
_BLK = 2304


def _vq_kernel(nblocks, n_rows, x_ref, embt_ref, embt2_bf_ref,
               inds_ref, cp_ref, pp_ref,
               e2_ref, psum_ref, hist_ref, riota_ref):
    i = pl.program_id(0)

    @pl.when(i == 0)
    def _init():
        embt = embt_ref[...]
        e2_ref[...] = jnp.sum(embt * embt, axis=0, keepdims=True)
        psum_ref[...] = jnp.zeros_like(psum_ref)
        hist_ref[...] = jnp.zeros_like(hist_ref)
        iota_i = jax.lax.broadcasted_iota(jnp.int32, riota_ref.shape, 1)
        riota_ref[...] = (_M - iota_i).astype(jnp.float32)

    x = x_ref[...]
    x2 = jnp.sum(x * x, axis=1, keepdims=True)
    s2 = jnp.dot(x.astype(jnp.bfloat16), embt2_bf_ref[...],
                 preferred_element_type=jnp.float32)
    dmap = s2 - (e2_ref[...] + x2)

    m = jnp.max(dmap, axis=1, keepdims=True)
    masked = jnp.where(dmap == m, riota_ref[...], 0.0)
    r = jnp.max(masked, axis=1, keepdims=True)
    k = (float(_M) - r).astype(jnp.int32)
    inds_ref[...] = k

    p = jnp.exp(dmap - m)
    inv = 1.0 / jnp.sum(p, axis=1, keepdims=True)
    ones_row = jnp.ones((1, x.shape[0]), jnp.bfloat16)
    pb = (p * inv).astype(jnp.bfloat16)
    psum_ref[...] += jnp.dot(ones_row, pb,
                             preferred_element_type=jnp.float32)

    ohb = jnp.where(masked == r, 1.0, 0.0).astype(jnp.bfloat16)
    hist_ref[...] += jnp.dot(ones_row, ohb,
                             preferred_element_type=jnp.float32)

    @pl.when(i == nblocks - 1)
    def _finish():
        inv_n = 1.0 / n_rows
        hp = hist_ref[...] * inv_n
        cp_ref[...] = -jnp.sum(hp * (jnp.log2(hp + 1e-10)), axis=1, keepdims=True)
        ap = psum_ref[...] * inv_n
        pp_ref[...] = -jnp.sum(ap * (jnp.log2(ap + 1e-10)), axis=1, keepdims=True)


def _sc_gather(n, chunk, table_hbm, idx_hbm, out_hbm, idx_v,
               rows_a, rows_b, sem_a, sem_b):
    info = plsc.get_sparse_core_info()
    nw = info.num_cores * info.num_subcores
    b_per_w = n // nw
    nchunks = b_per_w // chunk
    wid = lax.axis_index("s") * info.num_cores + lax.axis_index("c")
    base = wid * b_per_w
    pltpu.sync_copy(idx_hbm.at[pl.ds(base, b_per_w)], idx_v)
    rows = [rows_a, rows_b]
    sems = [sem_a, sem_b]
    handles = [pltpu.async_copy(table_hbm.at[idx_v.at[pl.ds(0, chunk)]],
                                rows_a, sem_a), None]
    for c in range(nchunks):
        handles[c % 2].wait()
        if c + 1 < nchunks:
            handles[(c + 1) % 2] = pltpu.async_copy(
                table_hbm.at[idx_v.at[pl.ds((c + 1) * chunk, chunk)]],
                rows[(c + 1) % 2], sems[(c + 1) % 2])
        pltpu.sync_copy(rows[c % 2], out_hbm.at[pl.ds(base + c * chunk, chunk)])


def kernel(x, embedding):
    bsz, tsz, csz = x.shape
    n = bsz * tsz
    x_flat = x.reshape(n, csz)
    emb = embedding[0]
    embt = emb.T
    nblocks = n // _BLK

    inds, cp, pp = pl.pallas_call(
        functools.partial(_vq_kernel, nblocks, float(n)),
        grid=(nblocks,),
        in_specs=[
            pl.BlockSpec((_BLK, _D), lambda i: (i, 0)),
            pl.BlockSpec((_D, _M), lambda i: (0, 0)),
            pl.BlockSpec((_D, _M), lambda i: (0, 0)),
        ],
        out_specs=[
            pl.BlockSpec((_BLK, 1), lambda i: (i, 0)),
            pl.BlockSpec((1, 1), lambda i: (0, 0)),
            pl.BlockSpec((1, 1), lambda i: (0, 0)),
        ],
        out_shape=[
            jax.ShapeDtypeStruct((n, 1), jnp.int32),
            jax.ShapeDtypeStruct((1, 1), jnp.float32),
            jax.ShapeDtypeStruct((1, 1), jnp.float32),
        ],
        scratch_shapes=[
            pltpu.VMEM((1, _M), jnp.float32),
            pltpu.VMEM((1, _M), jnp.float32),
            pltpu.VMEM((1, _M), jnp.float32),
            pltpu.VMEM((1, _M), jnp.float32),
        ],
    )(x_flat, embt, (2.0 * embt).astype(jnp.bfloat16))

    table = emb.astype(jnp.bfloat16).astype(jnp.float32)
    mesh = plsc.VectorSubcoreMesh(core_axis_name="c", subcore_axis_name="s")
    chunk = 96
    q = pl.kernel(
        functools.partial(_sc_gather, n, chunk),
        mesh=mesh,
        out_type=jax.ShapeDtypeStruct((n, _D), jnp.float32),
        scratch_types=[
            pltpu.VMEM((n // 32,), jnp.int32),
            pltpu.VMEM((chunk, _D), jnp.float32),
            pltpu.VMEM((chunk, _D), jnp.float32),
            pltpu.SemaphoreType.DMA,
            pltpu.SemaphoreType.DMA,
        ],
    )(table, inds.reshape(n))

    quantized = q.reshape(bsz, tsz, csz)
    quantization_inds = inds.reshape(bsz, tsz, 1)
    return (quantized, cp[0, 0], pp[0, 0], quantization_inds)


# speedup vs baseline: 1.3132x; 1.0135x over previous
import functools

import jax
import jax.numpy as jnp
from jax import lax
from jax.experimental import pallas as pl
from jax.experimental.pallas import tpu as pltpu
from jax.experimental.pallas import tpu_sc as plsc

_M = 1024
_D = 256
_BLK = 3072


def _vq_kernel(nblocks, n_rows, x_ref, e2_ref, embt2_bf_ref,
               inds_ref, cp_ref, pp_ref,
               psum_ref, hist_ref):
    i = pl.program_id(0)

    @pl.when(i == 0)
    def _init():
        psum_ref[...] = jnp.zeros_like(psum_ref)
        hist_ref[...] = jnp.zeros_like(hist_ref)

    x = x_ref[...]
    x2 = jnp.sum(x * x, axis=1, keepdims=True)
    s2 = jnp.dot(x.astype(jnp.bfloat16), embt2_bf_ref[...],
                 preferred_element_type=jnp.float32)
    dmap = s2 - (e2_ref[...] + x2)

    m = jnp.max(dmap, axis=1, keepdims=True)
    iota_i = jax.lax.broadcasted_iota(jnp.int32, dmap.shape, 1)
    riota = (_M - iota_i).astype(jnp.float32)
    masked = jnp.where(dmap == m, riota, 0.0)
    r = jnp.max(masked, axis=1, keepdims=True)
    k = (float(_M) - r).astype(jnp.int32)
    inds_ref[...] = k

    p = jnp.exp(dmap - m)
    inv = 1.0 / jnp.sum(p, axis=1, keepdims=True)
    ones_row = jnp.ones((1, x.shape[0]), jnp.bfloat16)
    pb = (p * inv).astype(jnp.bfloat16)
    psum_ref[...] += jnp.dot(ones_row, pb,
                             preferred_element_type=jnp.float32)

    ohb = jnp.where(masked == r, 1.0, 0.0).astype(jnp.bfloat16)
    hist_ref[...] += jnp.dot(ones_row, ohb,
                             preferred_element_type=jnp.float32)

    @pl.when(i == nblocks - 1)
    def _finish():
        inv_n = 1.0 / n_rows
        hp = hist_ref[...] * inv_n
        cp_ref[...] = -jnp.sum(hp * (jnp.log2(hp + 1e-10)), axis=1, keepdims=True)
        ap = psum_ref[...] * inv_n
        pp_ref[...] = -jnp.sum(ap * (jnp.log2(ap + 1e-10)), axis=1, keepdims=True)


def _sc_gather(n, chunk, table_hbm, idx_hbm, out_hbm, idx_v,
               rows_a, rows_b, sem_a, sem_b):
    info = plsc.get_sparse_core_info()
    nw = info.num_cores * info.num_subcores
    b_per_w = n // nw
    nchunks = b_per_w // chunk
    wid = lax.axis_index("s") * info.num_cores + lax.axis_index("c")
    base = wid * b_per_w
    pltpu.sync_copy(idx_hbm.at[pl.ds(base, b_per_w)], idx_v)
    rows = [rows_a, rows_b]
    sems = [sem_a, sem_b]
    handles = [pltpu.async_copy(table_hbm.at[idx_v.at[pl.ds(0, chunk)]],
                                rows_a, sem_a), None]
    for c in range(nchunks):
        handles[c % 2].wait()
        if c + 1 < nchunks:
            handles[(c + 1) % 2] = pltpu.async_copy(
                table_hbm.at[idx_v.at[pl.ds((c + 1) * chunk, chunk)]],
                rows[(c + 1) % 2], sems[(c + 1) % 2])
        pltpu.sync_copy(rows[c % 2], out_hbm.at[pl.ds(base + c * chunk, chunk)])


def kernel(x, embedding):
    bsz, tsz, csz = x.shape
    n = bsz * tsz
    x_flat = x.reshape(n, csz)
    emb = embedding[0]
    embt = emb.T
    e2 = jnp.sum(emb * emb, axis=1)[None, :]
    nblocks = n // _BLK

    inds, cp, pp = pl.pallas_call(
        functools.partial(_vq_kernel, nblocks, float(n)),
        grid=(nblocks,),
        in_specs=[
            pl.BlockSpec((_BLK, _D), lambda i: (i, 0)),
            pl.BlockSpec((1, _M), lambda i: (0, 0)),
            pl.BlockSpec((_D, _M), lambda i: (0, 0)),
        ],
        out_specs=[
            pl.BlockSpec((_BLK, 1), lambda i: (i, 0)),
            pl.BlockSpec((1, 1), lambda i: (0, 0)),
            pl.BlockSpec((1, 1), lambda i: (0, 0)),
        ],
        out_shape=[
            jax.ShapeDtypeStruct((n, 1), jnp.int32),
            jax.ShapeDtypeStruct((1, 1), jnp.float32),
            jax.ShapeDtypeStruct((1, 1), jnp.float32),
        ],
        scratch_shapes=[
            pltpu.VMEM((1, _M), jnp.float32),
            pltpu.VMEM((1, _M), jnp.float32),
        ],
    )(x_flat, e2, (2.0 * embt).astype(jnp.bfloat16))

    table = emb.astype(jnp.bfloat16).astype(jnp.float32)
    mesh = plsc.VectorSubcoreMesh(core_axis_name="c", subcore_axis_name="s")
    chunk = 96
    q = pl.kernel(
        functools.partial(_sc_gather, n, chunk),
        mesh=mesh,
        out_type=jax.ShapeDtypeStruct((n, _D), jnp.float32),
        scratch_types=[
            pltpu.VMEM((n // 32,), jnp.int32),
            pltpu.VMEM((chunk, _D), jnp.float32),
            pltpu.VMEM((chunk, _D), jnp.float32),
            pltpu.SemaphoreType.DMA,
            pltpu.SemaphoreType.DMA,
        ],
    )(table, inds.reshape(n))

    quantized = q.reshape(bsz, tsz, csz)
    quantization_inds = inds.reshape(bsz, tsz, 1)
    return (quantized, cp[0, 0], pp[0, 0], quantization_inds)
